# baseline (speedup 1.0000x reference)
import functools

import jax
import jax.numpy as jnp
from jax import lax
from jax.experimental import pallas as pl
from jax.experimental.pallas import tpu as pltpu
from jax.experimental.pallas import tpu_sc as plsc

VOCAB = 1000000
DIM = 64
B = 4096
L = 200

NC = 2
NS = 16
NW = NC * NS

ROWS_W = B // NW
RCH = 2
N_OUTER = ROWS_W // RCH
SPLIT = 128
REST = L - SPLIT

_mesh = plsc.VectorSubcoreMesh(core_axis_name="c", subcore_axis_name="s")


@functools.partial(
    pl.kernel,
    mesh=_mesh,
    out_type=jax.ShapeDtypeStruct((B, L, DIM), jnp.float32),
    scratch_types=[
        pltpu.VMEM((RCH, L), jnp.int32),
        pltpu.VMEM((RCH, L, DIM), jnp.float32),
        pltpu.SemaphoreType.DMA,
    ],
    compiler_params=pltpu.CompilerParams(use_tc_tiling_on_sc=False),
)
def _emb_lookup(x_hbm, table_hbm, out_hbm, idx_v, rows_v, sem):
    wid = lax.axis_index("s") * NC + lax.axis_index("c")
    row0 = wid * ROWS_W

    def chunk(c, carry):
        b0 = row0 + c * RCH
        pltpu.sync_copy(x_hbm.at[pl.ds(b0, RCH)], idx_v)
        cps = []
        for j in range(RCH):
            cps.append(pltpu.async_copy(
                table_hbm.at[idx_v.at[j, pl.ds(0, SPLIT)]],
                rows_v.at[j, pl.ds(0, SPLIT)],
                sem,
            ))
            cps.append(pltpu.async_copy(
                table_hbm.at[idx_v.at[j, pl.ds(SPLIT, REST)]],
                rows_v.at[j, pl.ds(SPLIT, REST)],
                sem,
            ))
        for cp in cps:
            cp.wait()
        pltpu.sync_copy(rows_v, out_hbm.at[pl.ds(b0, RCH)])
        return carry

    lax.fori_loop(0, N_OUTER, chunk, 0)


def kernel(x, table):
    return _emb_lookup(x, table)
